# baseline (speedup 1.0000x reference)
import functools

import jax
import jax.numpy as jnp
from jax import lax
from jax.experimental import pallas as pl
from jax.experimental.pallas import tpu as pltpu
from jax.experimental.pallas import tpu_sc as plsc

_EPS = 1e-09

_GATHER_WINDOW = 128
_SEQS_PER_BLOCK = 16


def _sc_gather(token_table, flat_idx):
    n = flat_idx.shape[0]
    d = token_table.shape[1]
    mesh = plsc.VectorSubcoreMesh(core_axis_name="c", subcore_axis_name="s")
    num_windows = n // _GATHER_WINDOW

    @functools.partial(
        pl.kernel,
        out_type=jax.ShapeDtypeStruct((n, d), token_table.dtype),
        mesh=mesh,
        compiler_params=pltpu.CompilerParams(use_tc_tiling_on_sc=False),
    )
    def gather_kernel(table_hbm, idx_hbm, out_hbm):
        def body(idx_vmem, out_vmem):
            pltpu.sync_copy(table_hbm.at[idx_vmem.at[0]], out_vmem)

        pltpu.emit_pipeline(
            body,
            grid=(num_windows,),
            in_specs=[
                pl.BlockSpec((1, _GATHER_WINDOW), index_map=lambda i: (0, i))
            ],
            out_specs=[
                pl.BlockSpec((_GATHER_WINDOW, d), index_map=lambda i: (i, 0))
            ],
            core_axis_name=("c", "s"),
            dimension_semantics=(pltpu.PARALLEL,),
        )(idx_hbm, out_hbm)

    return gather_kernel(token_table, flat_idx.reshape(1, n))


def _ln_body(seq_len, num_seqs, h_ref, pos_ref, b_ref, out_ref):
    bvec = b_ref[...]
    for s in range(num_seqs):
        rows = pl.ds(s * seq_len, seq_len)
        h = h_ref[rows, :] + pos_ref[...]
        mean = jnp.mean(h, axis=-1, keepdims=True)
        c = h - mean
        var = jnp.sum(c * c, axis=-1, keepdims=True) / (h.shape[-1] - 1)
        out_ref[rows, :] = c / (_EPS + jnp.sqrt(var)) + bvec


def _tc_layernorm(h_flat, pos, b):
    n, d = h_flat.shape
    seq_len = pos.shape[0]
    block_rows = seq_len * _SEQS_PER_BLOCK
    grid = n // block_rows
    return pl.pallas_call(
        functools.partial(_ln_body, seq_len, _SEQS_PER_BLOCK),
        grid=(grid,),
        in_specs=[
            pl.BlockSpec((block_rows, d), lambda i: (i, 0)),
            pl.BlockSpec((seq_len, d), lambda i: (0, 0)),
            pl.BlockSpec((1, d), lambda i: (0, 0)),
        ],
        out_specs=pl.BlockSpec((block_rows, d), lambda i: (i, 0)),
        out_shape=jax.ShapeDtypeStruct((n, d), jnp.float32),
    )(h_flat, pos, b.reshape(1, d))


def kernel(x, token_table, pos_table, a, b):
    batch, seq = x.shape
    d = token_table.shape[1]
    flat_idx = x.reshape(-1).astype(jnp.int32)
    gathered = _sc_gather(token_table, flat_idx)
    out = _tc_layernorm(gathered, pos_table[:seq], b)
    return out.reshape(batch, seq, d)


# speedup vs baseline: 1.0769x; 1.0769x over previous
import functools

import jax
import jax.numpy as jnp
from jax import lax
from jax.experimental import pallas as pl
from jax.experimental.pallas import tpu as pltpu
from jax.experimental.pallas import tpu_sc as plsc

_EPS = 1e-09
_SEQS_PER_BLOCK = 16


def _sc_gather(token_table, x2d):
    batch, seq = x2d.shape
    d = token_table.shape[1]
    mesh = plsc.VectorSubcoreMesh(core_axis_name="c", subcore_axis_name="s")

    num_workers = 32
    per_w = (batch // 2) // num_workers

    @functools.partial(
        pl.kernel,
        out_type=jax.ShapeDtypeStruct((batch // 2 * seq, 2 * d),
                                      token_table.dtype),
        mesh=mesh,
        scratch_types=[
            pltpu.VMEM((2, seq), jnp.int32),
            pltpu.VMEM((seq, d), token_table.dtype),
            pltpu.VMEM((seq, d), token_table.dtype),
            pltpu.SemaphoreType.DMA,
            pltpu.SemaphoreType.DMA,
            pltpu.SemaphoreType.DMA,
        ],
        compiler_params=pltpu.CompilerParams(use_tc_tiling_on_sc=False),
    )
    def gather_kernel(table_hbm, idx_hbm, out_hbm, idx_v, rows_a, rows_b,
                      sem_a, sem_b, sem_o):
        wid = lax.axis_index("s") * 2 + lax.axis_index("c")
        base = wid * per_w

        @pl.loop(0, per_w)
        def _(i):
            w = base + i
            pltpu.sync_copy(idx_hbm.at[pl.ds(2 * w, 2)], idx_v)
            ga = pltpu.async_copy(table_hbm.at[idx_v.at[0]], rows_a, sem_a)
            gb = pltpu.async_copy(table_hbm.at[idx_v.at[1]], rows_b, sem_b)
            ga.wait()
            oa = pltpu.async_copy(
                rows_a, out_hbm.at[pl.ds(w * seq, seq), pl.ds(0, d)], sem_o)
            gb.wait()
            ob = pltpu.async_copy(
                rows_b, out_hbm.at[pl.ds(w * seq, seq), pl.ds(d, d)], sem_o)
            oa.wait()
            ob.wait()

    return gather_kernel(token_table, x2d)


def _ln_body(seq, d, h_ref, pos_ref, b_ref, out_ref):
    pairs = _SEQS_PER_BLOCK // 2
    bvec = b_ref[...]
    sign = jnp.where(
        lax.broadcasted_iota(jnp.int32, (seq, 2 * d), 1) < d, 1.0, -1.0
    )
    for p in range(pairs):
        h = h_ref[pl.ds(p * seq, seq), :] + pos_ref[...]
        s_all = jnp.sum(h, axis=-1, keepdims=True)
        s_sgn = jnp.sum(h * sign, axis=-1, keepdims=True)
        mean = (s_all + sign * s_sgn) * (0.5 / d)
        c = h - mean
        sq = c * c
        v_all = jnp.sum(sq, axis=-1, keepdims=True)
        v_sgn = jnp.sum(sq * sign, axis=-1, keepdims=True)
        var_sum = (v_all + sign * v_sgn) * 0.5
        std = jnp.sqrt(var_sum * (1.0 / (d - 1)))
        out = c / (_EPS + std) + bvec
        out_ref[2 * p, :, :] = out[:, :d]
        out_ref[2 * p + 1, :, :] = out[:, d:]


def _tc_layernorm(h_pair, pos_pair, b_pair, batch, seq, d):
    pairs = _SEQS_PER_BLOCK // 2
    block_rows = seq * pairs
    grid = batch // _SEQS_PER_BLOCK
    return pl.pallas_call(
        functools.partial(_ln_body, seq, d),
        grid=(grid,),
        in_specs=[
            pl.BlockSpec((block_rows, 2 * d), lambda i: (i, 0)),
            pl.BlockSpec((seq, 2 * d), lambda i: (0, 0)),
            pl.BlockSpec((1, 2 * d), lambda i: (0, 0)),
        ],
        out_specs=pl.BlockSpec(
            (_SEQS_PER_BLOCK, seq, d), lambda i: (i, 0, 0)
        ),
        out_shape=jax.ShapeDtypeStruct((batch, seq, d), jnp.float32),
    )(h_pair, pos_pair, b_pair)


def kernel(x, token_table, pos_table, a, b):
    batch, seq = x.shape
    d = token_table.shape[1]
    gathered = _sc_gather(token_table, x.astype(jnp.int32))
    pos = pos_table[:seq]
    pos_pair = jnp.concatenate([pos, pos], axis=1)
    b_pair = jnp.concatenate([b, b]).reshape(1, 2 * d)
    return _tc_layernorm(gathered, pos_pair, b_pair, batch, seq, d)


# speedup vs baseline: 1.1468x; 1.0649x over previous
import functools

import jax
import jax.numpy as jnp
from jax import lax
from jax.experimental import pallas as pl
from jax.experimental.pallas import tpu as pltpu
from jax.experimental.pallas import tpu_sc as plsc

_EPS = 1e-09
_SEQS_PER_BLOCK = 16


def _sc_gather(token_table, x2d):
    batch, seq = x2d.shape
    d = token_table.shape[1]
    mesh = plsc.VectorSubcoreMesh(core_axis_name="c", subcore_axis_name="s")

    num_workers = 32
    per_w = (batch // 2) // num_workers

    @functools.partial(
        pl.kernel,
        out_type=jax.ShapeDtypeStruct((batch // 2 * seq, 2 * d),
                                      token_table.dtype),
        mesh=mesh,
        scratch_types=[
            pltpu.VMEM((2 * seq,), jnp.int32),
            pltpu.VMEM((seq, d), token_table.dtype),
            pltpu.VMEM((seq, d), token_table.dtype),
            pltpu.SemaphoreType.DMA,
            pltpu.SemaphoreType.DMA,
            pltpu.SemaphoreType.DMA,
        ],
        compiler_params=pltpu.CompilerParams(use_tc_tiling_on_sc=False),
    )
    def gather_kernel(table_hbm, idx_hbm, out_hbm, idx_v, rows_a, rows_b,
                      sem_a, sem_b, sem_o):
        wid = lax.axis_index("s") * 2 + lax.axis_index("c")
        base = wid * per_w

        @pl.loop(0, per_w)
        def _(i):
            w = base + i
            pltpu.sync_copy(idx_hbm.at[pl.ds(2 * seq * w, 2 * seq)], idx_v)
            ga = pltpu.async_copy(
                table_hbm.at[idx_v.at[pl.ds(0, seq)]], rows_a, sem_a)
            gb = pltpu.async_copy(
                table_hbm.at[idx_v.at[pl.ds(seq, seq)]], rows_b, sem_b)
            ga.wait()
            oa = pltpu.async_copy(
                rows_a, out_hbm.at[pl.ds(w * seq, seq), pl.ds(0, d)], sem_o)
            gb.wait()
            ob = pltpu.async_copy(
                rows_b, out_hbm.at[pl.ds(w * seq, seq), pl.ds(d, d)], sem_o)
            oa.wait()
            ob.wait()

    return gather_kernel(token_table, x2d.reshape(-1))


def _ln_body(seq, d, h_ref, pos_ref, b_ref, out_ref):
    pairs = _SEQS_PER_BLOCK // 2
    bvec = b_ref[...]
    sign = jnp.where(
        lax.broadcasted_iota(jnp.int32, (seq, 2 * d), 1) < d, 1.0, -1.0
    )
    for p in range(pairs):
        h = h_ref[pl.ds(p * seq, seq), :] + pos_ref[...]
        s_all = jnp.sum(h, axis=-1, keepdims=True)
        s_sgn = jnp.sum(h * sign, axis=-1, keepdims=True)
        mean = (s_all + sign * s_sgn) * (0.5 / d)
        c = h - mean
        sq = c * c
        v_all = jnp.sum(sq, axis=-1, keepdims=True)
        v_sgn = jnp.sum(sq * sign, axis=-1, keepdims=True)
        var_sum = (v_all + sign * v_sgn) * 0.5
        std = jnp.sqrt(var_sum * (1.0 / (d - 1)))
        out = c / (_EPS + std) + bvec
        out_ref[pl.ds(2 * p * seq, seq), :] = out[:, :d]
        out_ref[pl.ds((2 * p + 1) * seq, seq), :] = out[:, d:]


def _tc_layernorm(h_pair, pos_pair, b_pair, batch, seq, d):
    pairs = _SEQS_PER_BLOCK // 2
    block_rows = seq * pairs
    grid = batch // _SEQS_PER_BLOCK
    return pl.pallas_call(
        functools.partial(_ln_body, seq, d),
        grid=(grid,),
        in_specs=[
            pl.BlockSpec((block_rows, 2 * d), lambda i: (i, 0)),
            pl.BlockSpec((seq, 2 * d), lambda i: (0, 0)),
            pl.BlockSpec((1, 2 * d), lambda i: (0, 0)),
        ],
        out_specs=pl.BlockSpec(
            (_SEQS_PER_BLOCK * seq, d), lambda i: (i, 0)
        ),
        out_shape=jax.ShapeDtypeStruct((batch * seq, d), jnp.float32),
    )(h_pair, pos_pair, b_pair)


def kernel(x, token_table, pos_table, a, b):
    batch, seq = x.shape
    d = token_table.shape[1]
    gathered = _sc_gather(token_table, x.astype(jnp.int32))
    pos = pos_table[:seq]
    pos_pair = jnp.concatenate([pos, pos], axis=1)
    b_pair = jnp.concatenate([b, b]).reshape(1, 2 * d)
    out = _tc_layernorm(gathered, pos_pair, b_pair, batch, seq, d)
    return out.reshape(batch, seq, d)


# speedup vs baseline: 1.1613x; 1.0127x over previous
import functools

import jax
import jax.numpy as jnp
from jax import lax
from jax.experimental import pallas as pl
from jax.experimental.pallas import tpu as pltpu
from jax.experimental.pallas import tpu_sc as plsc

_EPS = 1e-09
_GATHER_ROWS = 128
_SLABS_PER_BLOCK = 16


def _sc_gather(token_table, idx2d, seq):
    d = token_table.shape[1]
    n = idx2d.shape[0] * idx2d.shape[1]
    num_workers = 32
    half_rows = idx2d.shape[0] // num_workers // 2
    steps = half_rows
    mesh = plsc.VectorSubcoreMesh(core_axis_name="c", subcore_axis_name="s")

    @functools.partial(
        pl.kernel,
        out_type=jax.ShapeDtypeStruct((n // 2, 2 * d), token_table.dtype),
        mesh=mesh,
        scratch_types=[
            pltpu.VMEM((2 * half_rows, _GATHER_ROWS), jnp.int32),
            pltpu.VMEM((_GATHER_ROWS, d), token_table.dtype),
            pltpu.VMEM((_GATHER_ROWS, d), token_table.dtype),
            pltpu.SemaphoreType.DMA,
            pltpu.SemaphoreType.DMA,
            pltpu.SemaphoreType.DMA,
        ],
        compiler_params=pltpu.CompilerParams(use_tc_tiling_on_sc=False),
    )
    def gather_kernel(table_hbm, idx_hbm, out_hbm, idx_v, rows_a, rows_b,
                      sem_a, sem_b, sem_o):
        wid = lax.axis_index("s") * 2 + lax.axis_index("c")
        idx_row0 = wid * 2 * half_rows
        out_row0 = wid * half_rows * _GATHER_ROWS
        pltpu.sync_copy(idx_hbm.at[pl.ds(idx_row0, 2 * half_rows)], idx_v)

        @pl.loop(0, steps)
        def _(k):
            ga = pltpu.async_copy(
                table_hbm.at[idx_v.at[k]], rows_a, sem_a)
            gb = pltpu.async_copy(
                table_hbm.at[idx_v.at[half_rows + k]], rows_b, sem_b)
            out_rows = pl.ds(out_row0 + k * _GATHER_ROWS, _GATHER_ROWS)
            ga.wait()
            oa = pltpu.async_copy(
                rows_a, out_hbm.at[out_rows, pl.ds(0, d)], sem_o)
            gb.wait()
            ob = pltpu.async_copy(
                rows_b, out_hbm.at[out_rows, pl.ds(d, d)], sem_o)
            oa.wait()
            ob.wait()

    return gather_kernel(token_table, idx2d)


def _ln_body(seq, d, h_ref, pos_ref, b_ref, out_ref):
    bvec = b_ref[...]
    sign = jnp.where(
        lax.broadcasted_iota(jnp.int32, (seq, 2 * d), 1) < d, 1.0, -1.0
    )
    for p in range(_SLABS_PER_BLOCK):
        h = h_ref[pl.ds(p * seq, seq), :] + pos_ref[...]
        s_all = jnp.sum(h, axis=-1, keepdims=True)
        s_sgn = jnp.sum(h * sign, axis=-1, keepdims=True)
        mean = (s_all + sign * s_sgn) * (0.5 / d)
        c = h - mean
        sq = c * c
        v_all = jnp.sum(sq, axis=-1, keepdims=True)
        v_sgn = jnp.sum(sq * sign, axis=-1, keepdims=True)
        var_sum = (v_all + sign * v_sgn) * 0.5
        std = jnp.sqrt(var_sum * (1.0 / (d - 1)))
        out = c / (_EPS + std) + bvec
        out_ref[0, pl.ds(p * seq, seq), :] = out[:, :d]
        out_ref[1, pl.ds(p * seq, seq), :] = out[:, d:]


def _tc_layernorm(h_pair, pos_pair, b_pair, batch, seq, d):
    block_rows = seq * _SLABS_PER_BLOCK
    n_half = h_pair.shape[0]
    grid = n_half // block_rows
    half_span = 4 * block_rows
    return pl.pallas_call(
        functools.partial(_ln_body, seq, d),
        grid=(grid,),
        in_specs=[
            pl.BlockSpec((block_rows, 2 * d), lambda i: (i, 0)),
            pl.BlockSpec((seq, 2 * d), lambda i: (0, 0)),
            pl.BlockSpec((1, 2 * d), lambda i: (0, 0)),
        ],
        out_specs=pl.BlockSpec(
            (2, block_rows, d), lambda i: (i // 4, i % 4, 0)
        ),
        out_shape=jax.ShapeDtypeStruct(
            (2 * n_half // half_span, half_span, d), jnp.float32),
    )(h_pair, pos_pair, b_pair)


def kernel(x, token_table, pos_table, a, b):
    batch, seq = x.shape
    d = token_table.shape[1]
    n = batch * seq
    idx2d = x.astype(jnp.int32).reshape(n // 128, 128)
    gathered = _sc_gather(token_table, idx2d, seq)
    pos = pos_table[:seq]
    pos_pair = jnp.concatenate([pos, pos], axis=1)
    b_pair = jnp.concatenate([b, b]).reshape(1, 2 * d)
    out = _tc_layernorm(gathered, pos_pair, b_pair, batch, seq, d)
    return out.reshape(batch, seq, d)


# speedup vs baseline: 1.2798x; 1.1021x over previous
import functools

import jax
import jax.numpy as jnp
from jax import lax
from jax.experimental import pallas as pl
from jax.experimental.pallas import tpu as pltpu
from jax.experimental.pallas import tpu_sc as plsc

_EPS = 1e-09
_GATHER_ROWS = 128
_SLABS_PER_BLOCK = 16


def _sc_gather(token_table, idx2d, seq):
    d = token_table.shape[1]
    n = idx2d.shape[0] * idx2d.shape[1]
    num_workers = 32
    half_rows = idx2d.shape[0] // num_workers // 2
    steps = half_rows
    mesh = plsc.VectorSubcoreMesh(core_axis_name="c", subcore_axis_name="s")

    @functools.partial(
        pl.kernel,
        out_type=jax.ShapeDtypeStruct((n // 2, 2 * d), token_table.dtype),
        mesh=mesh,
        scratch_types=[
            pltpu.VMEM((2 * half_rows, _GATHER_ROWS), jnp.int32),
            pltpu.VMEM((_GATHER_ROWS, d), token_table.dtype),
            pltpu.VMEM((_GATHER_ROWS, d), token_table.dtype),
            pltpu.SemaphoreType.DMA,
            pltpu.SemaphoreType.DMA,
            pltpu.SemaphoreType.DMA,
        ],
        compiler_params=pltpu.CompilerParams(use_tc_tiling_on_sc=False),
    )
    def gather_kernel(table_hbm, idx_hbm, out_hbm, idx_v, rows_a, rows_b,
                      sem_a, sem_b, sem_o):
        wid = lax.axis_index("s") * 2 + lax.axis_index("c")
        idx_row0 = wid * 2 * half_rows
        out_row0 = wid * half_rows * _GATHER_ROWS
        pltpu.sync_copy(idx_hbm.at[pl.ds(idx_row0, 2 * half_rows)], idx_v)

        @pl.loop(0, steps)
        def _(k):
            ga = pltpu.async_copy(
                table_hbm.at[idx_v.at[k]], rows_a, sem_a)
            gb = pltpu.async_copy(
                table_hbm.at[idx_v.at[half_rows + k]], rows_b, sem_b)
            out_rows = pl.ds(out_row0 + k * _GATHER_ROWS, _GATHER_ROWS)
            ga.wait()
            oa = pltpu.async_copy(
                rows_a, out_hbm.at[out_rows, pl.ds(0, d)], sem_o)
            gb.wait()
            ob = pltpu.async_copy(
                rows_b, out_hbm.at[out_rows, pl.ds(d, d)], sem_o)
            oa.wait()
            ob.wait()

    return gather_kernel(token_table, idx2d)


def _ln_body(seq, d, h_ref, pos_ref, b_ref, out_ref):
    bvec = b_ref[...]
    sign = jnp.where(
        lax.broadcasted_iota(jnp.int32, (seq, 2 * d), 1) < d, 1.0, -1.0
    )
    for p in range(_SLABS_PER_BLOCK):
        r0 = p * seq
        h = h_ref[pl.ds(r0, seq), :] + pos_ref[...]
        hs = h * sign
        hh = h * h
        hhs = hs * h
        s_all = jnp.sum(h, axis=-1, keepdims=True)
        s_sgn = jnp.sum(hs, axis=-1, keepdims=True)
        q_all = jnp.sum(hh, axis=-1, keepdims=True)
        q_sgn = jnp.sum(hhs, axis=-1, keepdims=True)
        mean = (s_all + sign * s_sgn) * (0.5 / d)
        qh = (q_all + sign * q_sgn) * 0.5
        var_sum = qh - mean * mean * d
        scale = lax.rsqrt(var_sum * (1.0 / (d - 1)) + _EPS * _EPS)
        out = (h - mean) * scale + bvec
        out_ref[0, pl.ds(r0, seq), :] = out[:, :d]
        out_ref[1, pl.ds(r0, seq), :] = out[:, d:]


def _tc_layernorm(h_pair, pos_pair, b_pair, batch, seq, d):
    block_rows = seq * _SLABS_PER_BLOCK
    n_half = h_pair.shape[0]
    grid = n_half // block_rows
    half_span = 4 * block_rows
    return pl.pallas_call(
        functools.partial(_ln_body, seq, d),
        grid=(grid,),
        in_specs=[
            pl.BlockSpec((block_rows, 2 * d), lambda i: (i, 0)),
            pl.BlockSpec((seq, 2 * d), lambda i: (0, 0)),
            pl.BlockSpec((1, 2 * d), lambda i: (0, 0)),
        ],
        out_specs=pl.BlockSpec(
            (2, block_rows, d), lambda i: (i // 4, i % 4, 0)
        ),
        out_shape=jax.ShapeDtypeStruct(
            (2 * n_half // half_span, half_span, d), jnp.float32),
    )(h_pair, pos_pair, b_pair)


def kernel(x, token_table, pos_table, a, b):
    batch, seq = x.shape
    d = token_table.shape[1]
    n = batch * seq
    idx2d = x.astype(jnp.int32).reshape(n // 128, 128)
    gathered = _sc_gather(token_table, idx2d, seq)
    pos = pos_table[:seq]
    pos_pair = jnp.concatenate([pos, pos], axis=1)
    b_pair = jnp.concatenate([b, b]).reshape(1, 2 * d)
    out = _tc_layernorm(gathered, pos_pair, b_pair, batch, seq, d)
    return out.reshape(batch, seq, d)
